# Initial kernel scaffold; baseline (speedup 1.0000x reference)
import jax
import jax.numpy as jnp
from jax.experimental import pallas as pl


def kernel(x, edge_index, batch, root_index, W1, b1, W2, b2):
    raise NotImplementedError("write your pallas kernel here")


# speedup vs baseline: 2.3024x; 2.3024x over previous
import functools
import jax
import jax.numpy as jnp
from jax import lax
from jax.experimental import pallas as pl
from jax.experimental.pallas import tpu as pltpu
from jax.experimental.pallas import tpu_sc as plsc

N_NODES = 10000
N_PAD = 10240
E_EDGES = 160000
E_PAD = 163840
F = 256
NG = 128
BLK = 512
NBLK = N_PAD // BLK


def _mm_body(x_ref, w_ref, o_ref):
    o_ref[...] = jnp.dot(x_ref[...], w_ref[...],
                         preferred_element_type=jnp.float32)


def _matmul(x, w):
    n, k = x.shape
    m = w.shape[1]
    return pl.pallas_call(
        _mm_body,
        grid=(n // BLK,),
        in_specs=[pl.BlockSpec((BLK, k), lambda i: (i, 0)),
                  pl.BlockSpec((k, m), lambda i: (0, 0))],
        out_specs=pl.BlockSpec((BLK, m), lambda i: (i, 0)),
        out_shape=jax.ShapeDtypeStruct((n, m), jnp.float32),
    )(x, w)


def _scale_body(h_ref, deg_ref, s_ref, dinv_ref):
    dv = lax.rsqrt(deg_ref[...] + 1.0)
    dinv_ref[...] = dv
    s_ref[...] = h_ref[...] * dv


def _scale(h1, degcol):
    return pl.pallas_call(
        _scale_body,
        grid=(NBLK,),
        in_specs=[pl.BlockSpec((BLK, F), lambda i: (i, 0)),
                  pl.BlockSpec((BLK, 1), lambda i: (i, 0))],
        out_specs=[pl.BlockSpec((BLK, F), lambda i: (i, 0)),
                   pl.BlockSpec((BLK, 1), lambda i: (i, 0))],
        out_shape=[jax.ShapeDtypeStruct((N_PAD, F), jnp.float32),
                   jax.ShapeDtypeStruct((N_PAD, 1), jnp.float32)],
    )(h1, degcol)


def _root_mm_body(xr_ref, w_ref, o_ref):
    o_ref[...] = jnp.dot(jnp.maximum(xr_ref[...], 0.0), w_ref[...],
                         preferred_element_type=jnp.float32)


def _root_mm(xr, w2b):
    return pl.pallas_call(
        _root_mm_body,
        out_shape=jax.ShapeDtypeStruct((NG, F), jnp.float32),
    )(xr, w2b)


def _layer2_body(a1_ref, s1_ref, dinv_ref, b1_ref, batch_ref, w2a_ref, rr_ref,
                 x2_ref, s2_ref):
    dv = dinv_ref[...]
    x2 = dv * (a1_ref[...] + s1_ref[...]) + b1_ref[...]
    x2_ref[...] = x2
    z = jnp.maximum(x2, 0.0)
    bb = batch_ref[0, 0, :]
    oh = (bb[:, None] == lax.broadcasted_iota(jnp.int32, (BLK, NG), 1)
          ).astype(jnp.float32)
    h2 = (jnp.dot(z, w2a_ref[...], preferred_element_type=jnp.float32)
          + jnp.dot(oh, rr_ref[...], preferred_element_type=jnp.float32))
    s2_ref[...] = h2 * dv


def _layer2(a1, s1, dinv, b1row, batch3, w2a, rr):
    return pl.pallas_call(
        _layer2_body,
        grid=(NBLK,),
        in_specs=[pl.BlockSpec((BLK, F), lambda i: (i, 0)),
                  pl.BlockSpec((BLK, F), lambda i: (i, 0)),
                  pl.BlockSpec((BLK, 1), lambda i: (i, 0)),
                  pl.BlockSpec((1, F), lambda i: (0, 0)),
                  pl.BlockSpec((1, 1, BLK), lambda i: (i, 0, 0)),
                  pl.BlockSpec((F, F), lambda i: (0, 0)),
                  pl.BlockSpec((NG, F), lambda i: (0, 0))],
        out_specs=[pl.BlockSpec((BLK, F), lambda i: (i, 0)),
                   pl.BlockSpec((BLK, F), lambda i: (i, 0))],
        out_shape=[jax.ShapeDtypeStruct((N_PAD, F), jnp.float32),
                   jax.ShapeDtypeStruct((N_PAD, F), jnp.float32)],
    )(a1, s1, dinv, b1row, batch3, w2a, rr)


def _final_body(a2_ref, s2_ref, dinv_ref, b2_ref, batch_ref, x2r_ref, out_ref,
                sums_ref, cnt_ref):
    i = pl.program_id(0)

    @pl.when(i == 0)
    def _():
        sums_ref[...] = jnp.zeros_like(sums_ref)
        cnt_ref[...] = jnp.zeros_like(cnt_ref)

    f = jnp.maximum(dinv_ref[...] * (a2_ref[...] + s2_ref[...]) + b2_ref[...],
                    0.0)
    bb = batch_ref[0, 0, :]
    oh = (lax.broadcasted_iota(jnp.int32, (NG, BLK), 0) == bb[None, :]
          ).astype(jnp.float32)
    sums_ref[...] += jnp.dot(oh, f, preferred_element_type=jnp.float32)
    cnt_ref[...] += jnp.dot(oh, jnp.ones((BLK, 1), jnp.float32),
                            preferred_element_type=jnp.float32)

    @pl.when(i == NBLK - 1)
    def _():
        cnt = cnt_ref[...]
        out_ref[:, :F] = sums_ref[...] / jnp.maximum(cnt, 1.0)
        out_ref[:, F:] = jnp.where(cnt > 0.0, x2r_ref[...], 0.0)


def _final(a2, s2, dinv, b2row, batch3, x2r):
    return pl.pallas_call(
        _final_body,
        grid=(NBLK,),
        in_specs=[pl.BlockSpec((BLK, F), lambda i: (i, 0)),
                  pl.BlockSpec((BLK, F), lambda i: (i, 0)),
                  pl.BlockSpec((BLK, 1), lambda i: (i, 0)),
                  pl.BlockSpec((1, F), lambda i: (0, 0)),
                  pl.BlockSpec((1, 1, BLK), lambda i: (i, 0, 0)),
                  pl.BlockSpec((NG, F), lambda i: (0, 0))],
        out_specs=pl.BlockSpec((NG, 2 * F), lambda i: (0, 0)),
        out_shape=jax.ShapeDtypeStruct((NG, 2 * F), jnp.float32),
        scratch_shapes=[pltpu.VMEM((NG, F), jnp.float32),
                        pltpu.VMEM((NG, 1), jnp.float32)],
    )(a2, s2, dinv, b2row, batch3, x2r)


def _deg_and_rootx(x_pad, dst, root_index):
    deg = jax.ops.segment_sum(jnp.ones((E_EDGES,), jnp.float32),
                              dst, num_segments=N_PAD)
    xr = jnp.take(x_pad, root_index, axis=0)
    return deg[:, None], xr


def _edge_agg(s, src, dst):
    msg = jnp.take(s, src, axis=0)
    return jax.ops.segment_sum(msg, dst, num_segments=N_PAD)


def kernel(x, edge_index, batch, root_index, W1, b1, W2, b2):
    x_pad = jnp.pad(x, ((0, N_PAD - N_NODES), (0, 0)))
    src = edge_index[0]
    dst = edge_index[1]
    batch_pad = jnp.pad(batch, (0, N_PAD - N_NODES),
                        constant_values=NG)
    batch3 = batch_pad.reshape(NBLK, 1, BLK)
    b1row = b1.reshape(1, F)
    b2row = b2.reshape(1, F)
    w2a = W2[:F]
    w2b = W2[F:]

    degcol, xr = _deg_and_rootx(x_pad, dst, root_index)
    h1 = _matmul(x_pad, W1)
    s1, dinv = _scale(h1, degcol)
    rr = _root_mm(xr, w2b)
    a1 = _edge_agg(s1, src, dst)
    x2, s2 = _layer2(a1, s1, dinv, b1row, batch3, w2a, rr)
    a2 = _edge_agg(s2, src, dst)
    x2r = jnp.take(x2, root_index, axis=0)
    return _final(a2, s2, dinv, b2row, batch3, x2r)
